# Initial kernel scaffold; baseline (speedup 1.0000x reference)
import jax
import jax.numpy as jnp
from jax.experimental import pallas as pl


def kernel(g, n_feat, e_feat, lin0_W, lin0_b, en_W1, en_b1, en_W2, en_b2, conv_b, bn_g, bn_b):
    raise NotImplementedError("write your pallas kernel here")


# speedup vs baseline: 27.5995x; 27.5995x over previous
import functools

import jax
import jax.numpy as jnp
from jax import lax
from jax.experimental import pallas as pl
from jax.experimental.pallas import tpu as pltpu
from jax.experimental.pallas import tpu_sc as plsc

N_NODES = 10000
N_EDGES = 160000
D_IN = 128
H = 16
D_EDGE = 16
E_HID = 64

NC = 2
NS = 16
NW = NC * NS
EPW = N_EDGES // NW
CW = 125
NCH = EPW // CW
QC = 8
QN = NCH // QC
QROWS = QC * CW
NPAD = 10240
NPS = NPAD // NS

_HP = jax.lax.Precision.HIGHEST
_DP = jax.lax.Precision.DEFAULT


def _mesh():
    return plsc.VectorSubcoreMesh(core_axis_name="c", subcore_axis_name="s",
                                  num_cores=NC, num_subcores=NS)


_SC_PARAMS = pltpu.CompilerParams(use_tc_tiling_on_sc=False)


def _gather1_body(idx_hbm, ta_hbm, oa_hbm, idx_v, ra_v, sem):
    c = lax.axis_index("c")
    s = lax.axis_index("s")
    wid = c * NS + s
    pltpu.sync_copy(idx_hbm.at[wid], idx_v)

    def q_body(q, carry):
        cps = []
        for j in range(QC):
            jj = q * QC + j
            cps.append(pltpu.async_copy(ta_hbm.at[idx_v.at[jj]],
                                        ra_v.at[pl.ds(j * CW, CW)], sem))
        for cp in cps:
            cp.wait()
        pltpu.sync_copy(ra_v, oa_hbm.at[pl.ds(wid * EPW + q * QROWS, QROWS)])
        return carry

    lax.fori_loop(0, QN, q_body, 0)


def _sc_gather1(idx2d, table):
    k = pl.kernel(
        _gather1_body,
        out_type=jax.ShapeDtypeStruct((N_EDGES, H), jnp.float32),
        mesh=_mesh(),
        compiler_params=_SC_PARAMS,
        scratch_types=[
            pltpu.VMEM((NCH, CW), jnp.int32),
            pltpu.VMEM((QROWS, H), jnp.float32),
            pltpu.SemaphoreType.DMA,
        ],
    )
    return k(idx2d, table)


def _gather2_body(idx_hbm, ta_hbm, tb_hbm, oa_hbm, ob_hbm, idx_v, ra_v, rb_v, sem):
    c = lax.axis_index("c")
    s = lax.axis_index("s")
    wid = c * NS + s
    pltpu.sync_copy(idx_hbm.at[wid], idx_v)

    def q_body(q, carry):
        cps = []
        for j in range(QC):
            jj = q * QC + j
            cps.append(pltpu.async_copy(ta_hbm.at[idx_v.at[jj]],
                                        ra_v.at[pl.ds(j * CW, CW)], sem))
            cps.append(pltpu.async_copy(tb_hbm.at[idx_v.at[jj]],
                                        rb_v.at[pl.ds(j * CW, CW)], sem))
        for cp in cps:
            cp.wait()
        base = wid * EPW + q * QROWS
        pltpu.sync_copy(ra_v, oa_hbm.at[pl.ds(base, QROWS)])
        pltpu.sync_copy(rb_v, ob_hbm.at[pl.ds(base, QROWS)])
        return carry

    lax.fori_loop(0, QN, q_body, 0)


def _sc_gather2(idx2d, table_a, table_b):
    k = pl.kernel(
        _gather2_body,
        out_type=(jax.ShapeDtypeStruct((N_EDGES, H), jnp.float32),
                  jax.ShapeDtypeStruct((N_EDGES, H), jnp.float32)),
        mesh=_mesh(),
        compiler_params=_SC_PARAMS,
        scratch_types=[
            pltpu.VMEM((NCH, CW), jnp.int32),
            pltpu.VMEM((QROWS, H), jnp.float32),
            pltpu.VMEM((QROWS, H), jnp.float32),
            pltpu.SemaphoreType.DMA,
        ],
    )
    return k(idx2d, table_a, table_b)


def _scatter_body(idx_hbm, msg_hbm, oa_hbm, ob_hbm, idx_v, rows_v, sl_v, agg_sh, sem):
    c = lax.axis_index("c")
    s = lax.axis_index("s")
    wid = c * NS + s

    def z_body(i, carry):
        sl_v[i, :] = jnp.zeros((H,), jnp.float32)
        return carry

    lax.fori_loop(0, NPS, z_body, 0)
    pltpu.sync_copy(sl_v, agg_sh.at[pl.ds(s * NPS, NPS)])
    plsc.subcore_barrier()

    pltpu.sync_copy(idx_hbm.at[wid], idx_v)
    pltpu.sync_copy(msg_hbm.at[pl.ds(wid * EPW, EPW)], rows_v)

    def q_body(q, carry):
        for j in range(QC):
            jj = q * QC + j
            pltpu.sync_copy(rows_v.at[pl.ds(jj * CW, CW)],
                            agg_sh.at[idx_v.at[jj]], add=True)
        return carry

    lax.fori_loop(0, QN, q_body, 0)
    plsc.subcore_barrier()

    pltpu.sync_copy(agg_sh.at[pl.ds(s * NPS, NPS)], sl_v)

    @pl.when(c == 0)
    def _():
        pltpu.sync_copy(sl_v, oa_hbm.at[pl.ds(s * NPS, NPS)])

    @pl.when(c == 1)
    def _():
        pltpu.sync_copy(sl_v, ob_hbm.at[pl.ds(s * NPS, NPS)])


def _sc_scatter(idx2d, msg):
    k = pl.kernel(
        _scatter_body,
        out_type=(jax.ShapeDtypeStruct((NPAD, H), jnp.float32),
                  jax.ShapeDtypeStruct((NPAD, H), jnp.float32)),
        mesh=_mesh(),
        compiler_params=_SC_PARAMS,
        scratch_types=[
            pltpu.VMEM((NCH, CW), jnp.int32),
            pltpu.VMEM((EPW, H), jnp.float32),
            pltpu.VMEM((NPS, H), jnp.float32),
            pltpu.VMEM_SHARED((NPAD, H), jnp.float32),
            pltpu.SemaphoreType.DMA,
        ],
    )
    return k(idx2d, msg)


BN0 = 2000


def _lin0_body(nf_ref, wt_ref, b_ref, out_ref):
    out_ref[:] = jnp.maximum(
        jnp.dot(nf_ref[:], wt_ref[:], precision=_DP,
                preferred_element_type=jnp.float32) + b_ref[:], 0.0)


def _tc_lin0(n_feat, lin0T, b):
    return pl.pallas_call(
        _lin0_body,
        grid=(N_NODES // BN0,),
        in_specs=[
            pl.BlockSpec((BN0, D_IN), lambda i: (i, 0)),
            pl.BlockSpec((D_IN, H), lambda i: (0, 0)),
            pl.BlockSpec((1, H), lambda i: (0, 0)),
        ],
        out_specs=pl.BlockSpec((BN0, H), lambda i: (i, 0)),
        out_shape=jax.ShapeDtypeStruct((N_NODES, H), jnp.float32),
    )(n_feat, lin0T, b)


BE = 4000


def _msg_from(h, ef, w1T, b1, w2T, b2, R, S):
    eh = jnp.maximum(
        jnp.dot(ef, w1T, precision=_DP, preferred_element_type=jnp.float32) + b1,
        0.0)
    w = jnp.dot(eh, w2T, precision=_DP, preferred_element_type=jnp.float32) + b2
    hrep = jnp.dot(h, R, precision=_HP, preferred_element_type=jnp.float32)
    return jnp.dot(hrep * w, S, precision=_HP, preferred_element_type=jnp.float32)


def _edge1_body(ef_ref, ha_ref, w1T_ref, b1_ref, w2T_ref, b2_ref, R_ref, S_ref,
                msg_ref):
    msg_ref[:] = _msg_from(ha_ref[:], ef_ref[:], w1T_ref[:], b1_ref[:],
                           w2T_ref[:], b2_ref[:], R_ref[:], S_ref[:])


def _edge2_body(ef_ref, ha_ref, hb_ref, cb_ref, w1T_ref, b1_ref, w2T_ref,
                b2_ref, R_ref, S_ref, msg_ref):
    h = jnp.maximum(ha_ref[:] + hb_ref[:] + cb_ref[:], 0.0)
    msg_ref[:] = _msg_from(h, ef_ref[:], w1T_ref[:], b1_ref[:], w2T_ref[:],
                           b2_ref[:], R_ref[:], S_ref[:])


def _w_specs():
    return [
        pl.BlockSpec((D_EDGE, E_HID), lambda i: (0, 0)),
        pl.BlockSpec((1, E_HID), lambda i: (0, 0)),
        pl.BlockSpec((E_HID, H * H), lambda i: (0, 0)),
        pl.BlockSpec((1, H * H), lambda i: (0, 0)),
        pl.BlockSpec((H, H * H), lambda i: (0, 0)),
        pl.BlockSpec((H * H, H), lambda i: (0, 0)),
    ]


def _tc_edge1(e_feat, ha, w1T, b1, w2T, b2, R, S):
    return pl.pallas_call(
        _edge1_body,
        grid=(N_EDGES // BE,),
        in_specs=[
            pl.BlockSpec((BE, D_EDGE), lambda i: (i, 0)),
            pl.BlockSpec((BE, H), lambda i: (i, 0)),
        ] + _w_specs(),
        out_specs=pl.BlockSpec((BE, H), lambda i: (i, 0)),
        out_shape=jax.ShapeDtypeStruct((N_EDGES, H), jnp.float32),
    )(e_feat, ha, w1T, b1, w2T, b2, R, S)


def _tc_edge2(e_feat, ha, hb, cb, w1T, b1, w2T, b2, R, S):
    return pl.pallas_call(
        _edge2_body,
        grid=(N_EDGES // BE,),
        in_specs=[
            pl.BlockSpec((BE, D_EDGE), lambda i: (i, 0)),
            pl.BlockSpec((BE, H), lambda i: (i, 0)),
            pl.BlockSpec((BE, H), lambda i: (i, 0)),
            pl.BlockSpec((1, H), lambda i: (0, 0)),
        ] + _w_specs(),
        out_specs=pl.BlockSpec((BE, H), lambda i: (i, 0)),
        out_shape=jax.ShapeDtypeStruct((N_EDGES, H), jnp.float32),
    )(e_feat, ha, hb, cb, w1T, b1, w2T, b2, R, S)


def _bn_body(a_ref, b_ref, cb_ref, g_ref, bb_ref, y_ref):
    valid = lax.broadcasted_iota(jnp.int32, (NPAD, H), 0) < N_NODES
    x = jnp.where(valid, jnp.maximum(a_ref[:] + b_ref[:] + cb_ref[:], 0.0), 0.0)
    inv_n = jnp.float32(1.0 / N_NODES)
    mean = jnp.sum(x, axis=0, keepdims=True) * inv_n
    xc = x - mean
    var = jnp.sum(jnp.where(valid, xc * xc, 0.0), axis=0, keepdims=True) * inv_n
    y = xc * lax.rsqrt(var + 1e-5) * g_ref[:] + bb_ref[:]
    y_ref[:] = y[:N_NODES, :]


def _tc_bn(agg_a, agg_b, cb, g2, b2):
    return pl.pallas_call(
        _bn_body,
        out_shape=jax.ShapeDtypeStruct((N_NODES, H), jnp.float32),
    )(agg_a, agg_b, cb, g2, b2)


def kernel(g, n_feat, e_feat, lin0_W, lin0_b, en_W1, en_b1, en_W2, en_b2,
           conv_b, bn_g, bn_b):
    src2d = g[0].astype(jnp.int32).reshape(NW, NCH, CW)
    dst2d = g[1].astype(jnp.int32).reshape(NW, NCH, CW)
    lin0T = lin0_W.T
    w1T = en_W1.T
    w2T = en_W2.T
    b1 = en_b1.reshape(1, E_HID)
    b2 = en_b2.reshape(1, H * H)
    cb = conv_b.reshape(1, H)
    lb = lin0_b.reshape(1, H)
    eye = jnp.eye(H, dtype=jnp.float32)
    R = jnp.repeat(eye, H, axis=1)
    S = jnp.tile(eye, (H, 1))

    out0 = _tc_lin0(n_feat, lin0T, lb)
    ha = _sc_gather1(src2d, out0)
    msg = _tc_edge1(e_feat, ha, w1T, b1, w2T, b2, R, S)
    agg_a, agg_b = _sc_scatter(dst2d, msg)
    for _ in range(2):
        ha, hb = _sc_gather2(src2d, agg_a, agg_b)
        msg = _tc_edge2(e_feat, ha, hb, cb, w1T, b1, w2T, b2, R, S)
        agg_a, agg_b = _sc_scatter(dst2d, msg)
    return _tc_bn(agg_a, agg_b, cb, bn_g.reshape(1, H), bn_b.reshape(1, H))


# speedup vs baseline: 43.0897x; 1.5612x over previous
import functools

import jax
import jax.numpy as jnp
from jax import lax
from jax.experimental import pallas as pl
from jax.experimental.pallas import tpu as pltpu
from jax.experimental.pallas import tpu_sc as plsc

N_NODES = 10000
N_EDGES = 160000
D_IN = 128
H = 16
D_EDGE = 16
E_HID = 64

NC = 2
NS = 16
NW = NC * NS
EPW = N_EDGES // NW
CW = 125
NCH = EPW // CW
QC = 8
QN = NCH // QC
QROWS = QC * CW
NPAD = 10240
NPS = NPAD // NS
PK = 8
EP8 = N_EDGES // PK
QP8 = QROWS // PK
WP8 = EPW // PK

_HP = jax.lax.Precision.HIGHEST
_DP = jax.lax.Precision.DEFAULT


def _mesh():
    return plsc.VectorSubcoreMesh(core_axis_name="c", subcore_axis_name="s",
                                  num_cores=NC, num_subcores=NS)


_SC_PARAMS = pltpu.CompilerParams(use_tc_tiling_on_sc=False)


def _gather1_body(idx_hbm, ta_hbm, oa_pk, idx_v, ra_v, sem):
    c = lax.axis_index("c")
    s = lax.axis_index("s")
    wid = c * NS + s
    pltpu.sync_copy(idx_hbm.at[wid], idx_v)

    def q_body(q, carry):
        cps = []
        for j in range(QC):
            jj = q * QC + j
            cps.append(pltpu.async_copy(ta_hbm.at[idx_v.at[jj]],
                                        ra_v.at[pl.ds(j * CW, CW)], sem))
        for cp in cps:
            cp.wait()
        pltpu.sync_copy(ra_v, oa_pk.at[pl.ds(wid * EPW + q * QROWS, QROWS)])
        return carry

    lax.fori_loop(0, QN, q_body, 0)


def _sc_gather1(idx2d, table):
    k = pl.kernel(
        _gather1_body,
        out_type=jax.ShapeDtypeStruct((N_EDGES, H), jnp.float32),
        mesh=_mesh(),
        compiler_params=_SC_PARAMS,
        scratch_types=[
            pltpu.VMEM((NCH, CW), jnp.int32),
            pltpu.VMEM((QROWS, H), jnp.float32),
            pltpu.SemaphoreType.DMA,
        ],
    )
    return k(idx2d, table)


def _gather2_body(idx_hbm, ta_hbm, tb_hbm, oa_pk, ob_pk, idx_v, ra_v, rb_v, sem):
    c = lax.axis_index("c")
    s = lax.axis_index("s")
    wid = c * NS + s
    pltpu.sync_copy(idx_hbm.at[wid], idx_v)

    def q_body(q, carry):
        cps = []
        for j in range(QC):
            jj = q * QC + j
            cps.append(pltpu.async_copy(ta_hbm.at[idx_v.at[jj]],
                                        ra_v.at[pl.ds(j * CW, CW)], sem))
            cps.append(pltpu.async_copy(tb_hbm.at[idx_v.at[jj]],
                                        rb_v.at[pl.ds(j * CW, CW)], sem))
        for cp in cps:
            cp.wait()
        base = wid * EPW + q * QROWS
        pltpu.sync_copy(ra_v, oa_pk.at[pl.ds(base, QROWS)])
        pltpu.sync_copy(rb_v, ob_pk.at[pl.ds(base, QROWS)])
        return carry

    lax.fori_loop(0, QN, q_body, 0)


def _sc_gather2(idx2d, table_a, table_b):
    k = pl.kernel(
        _gather2_body,
        out_type=(jax.ShapeDtypeStruct((N_EDGES, H), jnp.float32),
                  jax.ShapeDtypeStruct((N_EDGES, H), jnp.float32)),
        mesh=_mesh(),
        compiler_params=_SC_PARAMS,
        scratch_types=[
            pltpu.VMEM((NCH, CW), jnp.int32),
            pltpu.VMEM((QROWS, H), jnp.float32),
            pltpu.VMEM((QROWS, H), jnp.float32),
            pltpu.SemaphoreType.DMA,
        ],
    )
    return k(idx2d, table_a, table_b)


def _scatter_body(idx_hbm, msg_pk, oa_hbm, ob_hbm, idx_v, rows_v, sl_v, agg_sh, sem):
    c = lax.axis_index("c")
    s = lax.axis_index("s")
    wid = c * NS + s


    def z_body(i, carry):
        sl_v[i, :] = jnp.zeros((H,), jnp.float32)
        return carry

    lax.fori_loop(0, NPS, z_body, 0)
    pltpu.sync_copy(sl_v, agg_sh.at[pl.ds(s * NPS, NPS)])
    plsc.subcore_barrier()

    pltpu.sync_copy(idx_hbm.at[wid], idx_v)
    pltpu.sync_copy(msg_pk.at[pl.ds(wid * EPW, EPW)], rows_v)

    def q_body(q, carry):
        for j in range(QC):
            jj = q * QC + j
            pltpu.sync_copy(rows_v.at[pl.ds(jj * CW, CW)],
                            agg_sh.at[idx_v.at[jj]], add=True)
        return carry

    lax.fori_loop(0, QN, q_body, 0)
    plsc.subcore_barrier()

    pltpu.sync_copy(agg_sh.at[pl.ds(s * NPS, NPS)], sl_v)

    @pl.when(c == 0)
    def _():
        pltpu.sync_copy(sl_v, oa_hbm.at[pl.ds(s * NPS, NPS)])

    @pl.when(c == 1)
    def _():
        pltpu.sync_copy(sl_v, ob_hbm.at[pl.ds(s * NPS, NPS)])


def _sc_scatter(idx2d, msg):
    k = pl.kernel(
        _scatter_body,
        out_type=(jax.ShapeDtypeStruct((NPAD, H), jnp.float32),
                  jax.ShapeDtypeStruct((NPAD, H), jnp.float32)),
        mesh=_mesh(),
        compiler_params=_SC_PARAMS,
        scratch_types=[
            pltpu.VMEM((NCH, CW), jnp.int32),
            pltpu.VMEM((EPW, H), jnp.float32),
            pltpu.VMEM((NPS, H), jnp.float32),
            pltpu.VMEM_SHARED((NPAD, H), jnp.float32),
            pltpu.SemaphoreType.DMA,
        ],
    )
    return k(idx2d, msg)


BN0 = 2000


def _lin0_body(nf_ref, wt_ref, b_ref, out_ref):
    out_ref[:] = jnp.maximum(
        jnp.dot(nf_ref[:], wt_ref[:], precision=_DP,
                preferred_element_type=jnp.float32) + b_ref[:], 0.0)


def _tc_lin0(n_feat, lin0T, b):
    return pl.pallas_call(
        _lin0_body,
        grid=(N_NODES // BN0,),
        in_specs=[
            pl.BlockSpec((BN0, D_IN), lambda i: (i, 0)),
            pl.BlockSpec((D_IN, H), lambda i: (0, 0)),
            pl.BlockSpec((1, H), lambda i: (0, 0)),
        ],
        out_specs=pl.BlockSpec((BN0, H), lambda i: (i, 0)),
        out_shape=jax.ShapeDtypeStruct((N_NODES, H), jnp.float32),
    )(n_feat, lin0T, b)


B8 = 1000


def _msg_from(h, ef, w1b, b1, w2b, b2, Rb, Sb):
    eh = jnp.maximum(
        jnp.dot(ef, w1b, precision=_DP, preferred_element_type=jnp.float32) + b1,
        0.0)
    w = jnp.dot(eh, w2b, precision=_DP, preferred_element_type=jnp.float32) + b2
    hrep = jnp.dot(h, Rb, precision=_HP, preferred_element_type=jnp.float32)
    return jnp.dot(hrep * w, Sb, precision=_HP, preferred_element_type=jnp.float32)


def _edge1_body(ef_ref, ha_ref, w1b_ref, b1_ref, w2b_ref, b2_ref, Rb_ref,
                Sb_ref, msg_ref):
    msg_ref[:] = _msg_from(ha_ref[:], ef_ref[:], w1b_ref[:], b1_ref[:],
                           w2b_ref[:], b2_ref[:], Rb_ref[:], Sb_ref[:])


def _edge2_body(ef_ref, ha_ref, hb_ref, cb_ref, w1b_ref, b1_ref, w2b_ref,
                b2_ref, Rb_ref, Sb_ref, msg_ref):
    h = jnp.maximum(ha_ref[:] + hb_ref[:] + cb_ref[:], 0.0)
    msg_ref[:] = _msg_from(h, ef_ref[:], w1b_ref[:], b1_ref[:], w2b_ref[:],
                           b2_ref[:], Rb_ref[:], Sb_ref[:])


def _w_specs():
    return [
        pl.BlockSpec((PK * D_EDGE, PK * E_HID), lambda i: (0, 0)),
        pl.BlockSpec((1, PK * E_HID), lambda i: (0, 0)),
        pl.BlockSpec((PK * E_HID, PK * H * H), lambda i: (0, 0)),
        pl.BlockSpec((1, PK * H * H), lambda i: (0, 0)),
        pl.BlockSpec((PK * H, PK * H * H), lambda i: (0, 0)),
        pl.BlockSpec((PK * H * H, PK * H), lambda i: (0, 0)),
    ]


def _tc_edge1(ef_pk, ha, w1b, b1, w2b, b2, Rb, Sb):
    return pl.pallas_call(
        _edge1_body,
        grid=(EP8 // B8,),
        in_specs=[
            pl.BlockSpec((B8, PK * H), lambda i: (i, 0)),
            pl.BlockSpec((B8, PK * H), lambda i: (i, 0)),
        ] + _w_specs(),
        out_specs=pl.BlockSpec((B8, PK * H), lambda i: (i, 0)),
        out_shape=jax.ShapeDtypeStruct((EP8, PK * H), jnp.float32),
    )(ef_pk, ha, w1b, b1, w2b, b2, Rb, Sb)


def _tc_edge2(ef_pk, ha, hb, cbt, w1b, b1, w2b, b2, Rb, Sb):
    return pl.pallas_call(
        _edge2_body,
        grid=(EP8 // B8,),
        in_specs=[
            pl.BlockSpec((B8, PK * H), lambda i: (i, 0)),
            pl.BlockSpec((B8, PK * H), lambda i: (i, 0)),
            pl.BlockSpec((B8, PK * H), lambda i: (i, 0)),
            pl.BlockSpec((1, PK * H), lambda i: (0, 0)),
        ] + _w_specs(),
        out_specs=pl.BlockSpec((B8, PK * H), lambda i: (i, 0)),
        out_shape=jax.ShapeDtypeStruct((EP8, PK * H), jnp.float32),
    )(ef_pk, ha, hb, cbt, w1b, b1, w2b, b2, Rb, Sb)


def _bn_body(a_ref, b_ref, cb_ref, g_ref, bb_ref, y_ref):
    valid = lax.broadcasted_iota(jnp.int32, (NPAD, H), 0) < N_NODES
    x = jnp.where(valid, jnp.maximum(a_ref[:] + b_ref[:] + cb_ref[:], 0.0), 0.0)
    inv_n = jnp.float32(1.0 / N_NODES)
    mean = jnp.sum(x, axis=0, keepdims=True) * inv_n
    xc = x - mean
    var = jnp.sum(jnp.where(valid, xc * xc, 0.0), axis=0, keepdims=True) * inv_n
    y = xc * lax.rsqrt(var + 1e-5) * g_ref[:] + bb_ref[:]
    y_ref[:] = y[:N_NODES, :]


def _tc_bn(agg_a, agg_b, cb, g2, b2):
    return pl.pallas_call(
        _bn_body,
        out_shape=jax.ShapeDtypeStruct((N_NODES, H), jnp.float32),
    )(agg_a, agg_b, cb, g2, b2)


def kernel(g, n_feat, e_feat, lin0_W, lin0_b, en_W1, en_b1, en_W2, en_b2,
           conv_b, bn_g, bn_b):
    src2d = g[0].astype(jnp.int32).reshape(NW, NCH, CW)
    dst2d = g[1].astype(jnp.int32).reshape(NW, NCH, CW)
    lin0T = lin0_W.T
    lb = lin0_b.reshape(1, H)
    eye8 = jnp.eye(PK, dtype=jnp.float32)
    eyeH = jnp.eye(H, dtype=jnp.float32)
    R = jnp.repeat(eyeH, H, axis=1)
    S = jnp.tile(eyeH, (H, 1))
    w1b = jnp.kron(eye8, en_W1.T)
    w2b = jnp.kron(eye8, en_W2.T)
    Rb = jnp.kron(eye8, R)
    Sb = jnp.kron(eye8, S)
    b1t = jnp.tile(en_b1, PK).reshape(1, PK * E_HID)
    b2t = jnp.tile(en_b2, PK).reshape(1, PK * H * H)
    cbt = jnp.tile(conv_b, PK).reshape(1, PK * H)
    ef_pk = e_feat.reshape(EP8, PK * D_EDGE)

    out0 = _tc_lin0(n_feat, lin0T, lb)
    ha = _sc_gather1(src2d, out0).reshape(EP8, PK * H)
    msg = _tc_edge1(ef_pk, ha, w1b, b1t, w2b, b2t, Rb, Sb)
    agg_a, agg_b = _sc_scatter(dst2d, msg.reshape(N_EDGES, H))
    for _ in range(2):
        ha, hb = _sc_gather2(src2d, agg_a, agg_b)
        ha = ha.reshape(EP8, PK * H)
        hb = hb.reshape(EP8, PK * H)
        msg = _tc_edge2(ef_pk, ha, hb, cbt, w1b, b1t, w2b, b2t, Rb, Sb)
        agg_a, agg_b = _sc_scatter(dst2d, msg.reshape(N_EDGES, H))
    cb = conv_b.reshape(1, H)
    return _tc_bn(agg_a, agg_b, cb, bn_g.reshape(1, H), bn_b.reshape(1, H))
